# baseline (speedup 1.0000x reference)
import math

import jax
import jax.numpy as jnp
from jax.experimental import pallas as pl
from jax.experimental.pallas import tpu as pltpu

_TOKEN_DIM = 192
_NUM_HEADS = 4
_HEAD_DIM = _TOKEN_DIM // _NUM_HEADS
_TOPK = 128

def _pos_encoding_t(Hs, Ws):
    y_pos = jnp.repeat(jnp.arange(Hs), Ws).astype(jnp.float32)
    x_pos = jnp.tile(jnp.arange(Ws), Hs).astype(jnp.float32)
    dim_half = _TOKEN_DIM // 2
    div_term = jnp.exp(
        jnp.arange(dim_half).astype(jnp.float32) * (-math.log(10000.0) / dim_half))
    pe_sin = jnp.sin(div_term[:, None] * x_pos[None, :])
    pe_cos = jnp.cos(div_term[: _TOKEN_DIM - dim_half, None] * y_pos[None, :])
    return jnp.concatenate([pe_sin, pe_cos], axis=0)


def _dot_t(a, b):
    return jax.lax.dot_general(
        a, b, (((0,), (0,)), ((), ())),
        preferred_element_type=jnp.float32,
        precision=jax.lax.Precision.DEFAULT)


def _dot_nt(a, b):
    return jax.lax.dot_general(
        a, b, (((1,), (1,)), ((), ())),
        preferred_element_type=jnp.float32,
        precision=jax.lax.Precision.DEFAULT)


def _proj_kernel(f1_ref, f2_ref, wq_ref, wk_ref, wv_ref,
                 bq_ref, bk_ref, bv_ref, pe_ref,
                 qt_ref, kt_ref, vt_ref):
    f1 = f1_ref[0]
    f2 = f2_ref[0]
    pe = pe_ref[...]
    qt_ref[0] = _dot_t(wq_ref[...], f1) + bq_ref[...] + pe
    kt_ref[0] = _dot_t(wk_ref[...], f2) + bk_ref[...] + pe
    vt_ref[0] = _dot_t(wv_ref[...], f2) + bv_ref[...]


def _sortable_keys(s):
    u = jax.lax.bitcast_convert_type(s, jnp.int32)
    return jnp.where(u >= 0, u, jnp.full_like(u, -2147483648) - u)


def _topk_mask(s, k):
    keys = _sortable_keys(s)
    n = keys.shape[-1]
    lo = jnp.min(keys, axis=-1, keepdims=True)
    hi = jnp.max(keys, axis=-1, keepdims=True)
    clo = jnp.full_like(lo, n)
    chi = jnp.zeros_like(lo)

    def cond(c):
        lo, hi, clo, chi = c
        return jnp.any(lo < hi)

    def body(c):
        lo, hi, clo, chi = c
        lo_f = lo.astype(jnp.float32)
        span = hi.astype(jnp.float32) - lo_f
        denom = jnp.maximum((clo - chi).astype(jnp.float32), 1.0)
        frac = (clo - k).astype(jnp.float32) / denom
        mid = jnp.clip((lo_f + frac * span).astype(jnp.int32), lo + 1, hi)
        cnt = jnp.sum((keys >= mid).astype(jnp.int32), axis=-1, keepdims=True)
        ge = cnt >= k
        eqk = cnt == k
        lo2 = jnp.where(ge, mid, lo)
        hi2 = jnp.where(eqk, mid, jnp.where(ge, hi, mid - 1))
        clo2 = jnp.where(ge, cnt, clo)
        chi2 = jnp.where(ge, chi, cnt)
        return lo2, hi2, clo2, chi2

    lo, hi, clo, chi = jax.lax.while_loop(cond, body, (lo, hi, clo, chi))
    return keys >= lo


def _attn_kernel(qt_ref, kt_ref, vt_ref, wo_ref, bo_ref, out_ref):
    scale = 1.0 / math.sqrt(_HEAD_DIM)
    ctx_parts = []
    for h in range(_NUM_HEADS):
        sl = slice(h * _HEAD_DIM, (h + 1) * _HEAD_DIM)
        q = qt_ref[0, sl, :]
        kk = kt_ref[0, sl, :]
        v = vt_ref[0, sl, :]
        s = _dot_t(q, kk) * scale
        mask = _topk_mask(s, _TOPK)
        rowmax = jnp.max(s, axis=-1, keepdims=True)
        w = jnp.where(mask, jnp.exp(s - rowmax), 0.0)
        denom = jnp.sum(w, axis=-1, keepdims=True)
        attn = w / denom
        ctx_parts.append(_dot_nt(v, attn))

    ctxT = jnp.concatenate(ctx_parts, axis=0)
    out_ref[0] = _dot_t(wo_ref[...], ctxT) + bo_ref[...]


def kernel(feat1, feat2, Wq, bq, Wk, bk, Wv, bv, Wo, bo):
    B, C, Hs, Ws = feat1.shape
    N = Hs * Ws
    f1 = feat1.reshape(B, C, N)
    f2 = feat2.reshape(B, C, N)
    peT = _pos_encoding_t(Hs, Ws)
    bq2 = bq.reshape(-1, 1)
    bk2 = bk.reshape(-1, 1)
    bv2 = bv.reshape(-1, 1)
    bo2 = bo.reshape(-1, 1)

    grid_p = (B,)
    qkv_shape = jax.ShapeDtypeStruct((B, _TOKEN_DIM, N), jnp.float32)
    qt, kt, vt = pl.pallas_call(
        _proj_kernel,
        grid=grid_p,
        in_specs=[
            pl.BlockSpec((1, C, N), lambda b: (b, 0, 0)),
            pl.BlockSpec((1, C, N), lambda b: (b, 0, 0)),
            pl.BlockSpec((C, _TOKEN_DIM), lambda b: (0, 0)),
            pl.BlockSpec((C, _TOKEN_DIM), lambda b: (0, 0)),
            pl.BlockSpec((C, _TOKEN_DIM), lambda b: (0, 0)),
            pl.BlockSpec((_TOKEN_DIM, 1), lambda b: (0, 0)),
            pl.BlockSpec((_TOKEN_DIM, 1), lambda b: (0, 0)),
            pl.BlockSpec((_TOKEN_DIM, 1), lambda b: (0, 0)),
            pl.BlockSpec((_TOKEN_DIM, N), lambda b: (0, 0)),
        ],
        out_specs=[
            pl.BlockSpec((1, _TOKEN_DIM, N), lambda b: (b, 0, 0)),
            pl.BlockSpec((1, _TOKEN_DIM, N), lambda b: (b, 0, 0)),
            pl.BlockSpec((1, _TOKEN_DIM, N), lambda b: (b, 0, 0)),
        ],
        out_shape=[qkv_shape, qkv_shape, qkv_shape],
    )(f1, f2, Wq, Wk, Wv, bq2, bk2, bv2, peT)

    BQ = 256
    grid_a = (B, pl.cdiv(N, BQ))
    outT = pl.pallas_call(
        _attn_kernel,
        grid=grid_a,
        in_specs=[
            pl.BlockSpec((1, _TOKEN_DIM, BQ), lambda b, j: (b, 0, j)),
            pl.BlockSpec((1, _TOKEN_DIM, N), lambda b, j: (b, 0, 0)),
            pl.BlockSpec((1, _TOKEN_DIM, N), lambda b, j: (b, 0, 0)),
            pl.BlockSpec((_TOKEN_DIM, _TOKEN_DIM), lambda b, j: (0, 0)),
            pl.BlockSpec((_TOKEN_DIM, 1), lambda b, j: (0, 0)),
        ],
        out_specs=pl.BlockSpec((1, _TOKEN_DIM, BQ), lambda b, j: (b, 0, j)),
        out_shape=jax.ShapeDtypeStruct((B, _TOKEN_DIM, N), jnp.float32),
        compiler_params=pltpu.CompilerParams(
            dimension_semantics=("parallel", "parallel")),
    )(qt, kt, vt, Wo, bo2)

    return outT.reshape(B, _TOKEN_DIM, Hs, Ws)


# speedup vs baseline: 2.5716x; 2.5716x over previous
import math

import jax
import jax.numpy as jnp
from jax.experimental import pallas as pl
from jax.experimental.pallas import tpu as pltpu

_TOKEN_DIM = 192
_NUM_HEADS = 4
_HEAD_DIM = _TOKEN_DIM // _NUM_HEADS
_TOPK = 128

def _pos_encoding_t(Hs, Ws):
    y_pos = jnp.repeat(jnp.arange(Hs), Ws).astype(jnp.float32)
    x_pos = jnp.tile(jnp.arange(Ws), Hs).astype(jnp.float32)
    dim_half = _TOKEN_DIM // 2
    div_term = jnp.exp(
        jnp.arange(dim_half).astype(jnp.float32) * (-math.log(10000.0) / dim_half))
    pe_sin = jnp.sin(div_term[:, None] * x_pos[None, :])
    pe_cos = jnp.cos(div_term[: _TOKEN_DIM - dim_half, None] * y_pos[None, :])
    return jnp.concatenate([pe_sin, pe_cos], axis=0)


def _dot_t(a, b):
    return jax.lax.dot_general(
        a, b, (((0,), (0,)), ((), ())),
        preferred_element_type=jnp.float32,
        precision=jax.lax.Precision.DEFAULT)


def _dot_nt(a, b):
    return jax.lax.dot_general(
        a, b, (((1,), (1,)), ((), ())),
        preferred_element_type=jnp.float32,
        precision=jax.lax.Precision.DEFAULT)


def _proj_kernel(f1_ref, f2_ref, wq_ref, wk_ref, wv_ref,
                 bq_ref, bk_ref, bv_ref, pe_ref,
                 qt_ref, kt_ref, vt_ref):
    f1 = f1_ref[0]
    f2 = f2_ref[0]
    pe = pe_ref[...]
    qt_ref[0] = _dot_t(wq_ref[...], f1) + bq_ref[...] + pe
    kt_ref[0] = _dot_t(wk_ref[...], f2) + bk_ref[...] + pe
    vt_ref[0] = _dot_t(wv_ref[...], f2) + bv_ref[...]


def _sortable_keys(s):
    u = jax.lax.bitcast_convert_type(s, jnp.int32)
    return jnp.where(u >= 0, u, jnp.full_like(u, -2147483648) - u)


def _topk_mask(s, k):
    keys = _sortable_keys(s)
    lo = jnp.min(keys, axis=-1, keepdims=True)
    hi = jnp.max(keys, axis=-1, keepdims=True)

    def body(_, c):
        lo, hi = c
        mid = (lo >> 1) + (hi >> 1) + ((lo | hi) & 1)
        cnt = jnp.sum(keys >= mid, axis=-1, keepdims=True)
        ge = cnt >= k
        return jnp.where(ge, mid, lo), jnp.where(ge, hi, mid - 1)

    lo, hi = jax.lax.fori_loop(0, 32, body, (lo, hi))
    return keys >= lo


def _attn_kernel(qt_ref, kt_ref, vt_ref, wo_ref, bo_ref, out_ref):
    scale = 1.0 / math.sqrt(_HEAD_DIM)
    bq = qt_ref.shape[2]
    s_parts = []
    for h in range(_NUM_HEADS):
        sl = slice(h * _HEAD_DIM, (h + 1) * _HEAD_DIM)
        q = qt_ref[0, sl, :]
        kk = kt_ref[0, sl, :]
        s_parts.append(_dot_t(q, kk) * scale)

    s = jnp.concatenate(s_parts, axis=0)
    mask = _topk_mask(s, _TOPK)
    rowmax = jnp.max(s, axis=-1, keepdims=True)
    w = jnp.where(mask, jnp.exp(s - rowmax), 0.0)
    denom = jnp.sum(w, axis=-1, keepdims=True)
    attn = w / denom

    ctx_parts = []
    for h in range(_NUM_HEADS):
        sl = slice(h * _HEAD_DIM, (h + 1) * _HEAD_DIM)
        v = vt_ref[0, sl, :]
        ctx_parts.append(_dot_nt(v, attn[h * bq:(h + 1) * bq]))

    ctxT = jnp.concatenate(ctx_parts, axis=0)
    out_ref[0] = _dot_t(wo_ref[...], ctxT) + bo_ref[...]


def kernel(feat1, feat2, Wq, bq, Wk, bk, Wv, bv, Wo, bo):
    B, C, Hs, Ws = feat1.shape
    N = Hs * Ws
    f1 = feat1.reshape(B, C, N)
    f2 = feat2.reshape(B, C, N)
    peT = _pos_encoding_t(Hs, Ws)
    bq2 = bq.reshape(-1, 1)
    bk2 = bk.reshape(-1, 1)
    bv2 = bv.reshape(-1, 1)
    bo2 = bo.reshape(-1, 1)

    grid_p = (B,)
    qkv_shape = jax.ShapeDtypeStruct((B, _TOKEN_DIM, N), jnp.float32)
    qt, kt, vt = pl.pallas_call(
        _proj_kernel,
        grid=grid_p,
        in_specs=[
            pl.BlockSpec((1, C, N), lambda b: (b, 0, 0)),
            pl.BlockSpec((1, C, N), lambda b: (b, 0, 0)),
            pl.BlockSpec((C, _TOKEN_DIM), lambda b: (0, 0)),
            pl.BlockSpec((C, _TOKEN_DIM), lambda b: (0, 0)),
            pl.BlockSpec((C, _TOKEN_DIM), lambda b: (0, 0)),
            pl.BlockSpec((_TOKEN_DIM, 1), lambda b: (0, 0)),
            pl.BlockSpec((_TOKEN_DIM, 1), lambda b: (0, 0)),
            pl.BlockSpec((_TOKEN_DIM, 1), lambda b: (0, 0)),
            pl.BlockSpec((_TOKEN_DIM, N), lambda b: (0, 0)),
        ],
        out_specs=[
            pl.BlockSpec((1, _TOKEN_DIM, N), lambda b: (b, 0, 0)),
            pl.BlockSpec((1, _TOKEN_DIM, N), lambda b: (b, 0, 0)),
            pl.BlockSpec((1, _TOKEN_DIM, N), lambda b: (b, 0, 0)),
        ],
        out_shape=[qkv_shape, qkv_shape, qkv_shape],
    )(f1, f2, Wq, Wk, Wv, bq2, bk2, bv2, peT)

    BQ = 256
    grid_a = (B, pl.cdiv(N, BQ))
    outT = pl.pallas_call(
        _attn_kernel,
        grid=grid_a,
        in_specs=[
            pl.BlockSpec((1, _TOKEN_DIM, BQ), lambda b, j: (b, 0, j)),
            pl.BlockSpec((1, _TOKEN_DIM, N), lambda b, j: (b, 0, 0)),
            pl.BlockSpec((1, _TOKEN_DIM, N), lambda b, j: (b, 0, 0)),
            pl.BlockSpec((_TOKEN_DIM, _TOKEN_DIM), lambda b, j: (0, 0)),
            pl.BlockSpec((_TOKEN_DIM, 1), lambda b, j: (0, 0)),
        ],
        out_specs=pl.BlockSpec((1, _TOKEN_DIM, BQ), lambda b, j: (b, 0, j)),
        out_shape=jax.ShapeDtypeStruct((B, _TOKEN_DIM, N), jnp.float32),
        compiler_params=pltpu.CompilerParams(
            dimension_semantics=("parallel", "parallel")),
    )(qt, kt, vt, Wo, bo2)

    return outT.reshape(B, _TOKEN_DIM, Hs, Ws)


# speedup vs baseline: 2.6653x; 1.0364x over previous
import math

import jax
import jax.numpy as jnp
from jax.experimental import pallas as pl
from jax.experimental.pallas import tpu as pltpu

_TOKEN_DIM = 192
_NUM_HEADS = 4
_HEAD_DIM = _TOKEN_DIM // _NUM_HEADS
_TOPK = 128

def _pos_encoding_t(Hs, Ws):
    y_pos = jnp.repeat(jnp.arange(Hs), Ws).astype(jnp.float32)
    x_pos = jnp.tile(jnp.arange(Ws), Hs).astype(jnp.float32)
    dim_half = _TOKEN_DIM // 2
    div_term = jnp.exp(
        jnp.arange(dim_half).astype(jnp.float32) * (-math.log(10000.0) / dim_half))
    pe_sin = jnp.sin(div_term[:, None] * x_pos[None, :])
    pe_cos = jnp.cos(div_term[: _TOKEN_DIM - dim_half, None] * y_pos[None, :])
    return jnp.concatenate([pe_sin, pe_cos], axis=0)


def _dot_t(a, b):
    return jax.lax.dot_general(
        a, b, (((0,), (0,)), ((), ())),
        preferred_element_type=jnp.float32,
        precision=jax.lax.Precision.DEFAULT)


def _dot_nt(a, b):
    return jax.lax.dot_general(
        a, b, (((1,), (1,)), ((), ())),
        preferred_element_type=jnp.float32,
        precision=jax.lax.Precision.DEFAULT)


def _proj_kernel(f1_ref, f2_ref, wq_ref, wk_ref, wv_ref,
                 bq_ref, bk_ref, bv_ref, pe_ref,
                 qt_ref, kt_ref, vt_ref):
    f1 = f1_ref[0]
    f2 = f2_ref[0]
    pe = pe_ref[...]
    qt_ref[0] = _dot_t(wq_ref[...], f1) + bq_ref[...] + pe
    kt_ref[0] = _dot_t(wk_ref[...], f2) + bk_ref[...] + pe
    vt_ref[0] = _dot_t(wv_ref[...], f2) + bv_ref[...]


def _sortable_keys(s):
    u = jax.lax.bitcast_convert_type(s, jnp.int32)
    return jnp.where(u >= 0, u, jnp.full_like(u, -2147483648) - u)


def _topk_mask(s, k):
    keys = _sortable_keys(s)
    rows = s.shape[0]
    lo = jnp.full((rows, 1), -2147483648, dtype=jnp.int32)
    hi = jnp.full((rows, 1), 2147483647, dtype=jnp.int32)

    def body(_, c):
        lo, hi = c
        mid = (lo >> 1) + (hi >> 1) + ((lo | hi) & 1)
        cnt = jnp.sum(keys >= mid, axis=-1, keepdims=True)
        ge = cnt >= k
        return jnp.where(ge, mid, lo), jnp.where(ge, hi, mid - 1)

    lo, hi = jax.lax.fori_loop(0, 32, body, (lo, hi))
    return keys >= lo


def _attn_kernel(qt_ref, kt_ref, vt_ref, wo_ref, bo_ref, out_ref):
    scale = 1.0 / math.sqrt(_HEAD_DIM)
    bq = qt_ref.shape[2]
    s_parts = []
    for h in range(_NUM_HEADS):
        sl = slice(h * _HEAD_DIM, (h + 1) * _HEAD_DIM)
        q = qt_ref[0, sl, :]
        kk = kt_ref[0, sl, :]
        s_parts.append(_dot_t(q, kk) * scale)

    s = jnp.concatenate(s_parts, axis=0)
    mask = _topk_mask(s, _TOPK)
    rowmax = jnp.max(s, axis=-1, keepdims=True)
    w = jnp.where(mask, jnp.exp(s - rowmax), 0.0)
    denom = jnp.sum(w, axis=-1, keepdims=True)
    attn = w / denom

    ctx_parts = []
    for h in range(_NUM_HEADS):
        sl = slice(h * _HEAD_DIM, (h + 1) * _HEAD_DIM)
        v = vt_ref[0, sl, :]
        ctx_parts.append(_dot_nt(v, attn[h * bq:(h + 1) * bq]))

    ctxT = jnp.concatenate(ctx_parts, axis=0)
    out_ref[0] = _dot_t(wo_ref[...], ctxT) + bo_ref[...]


def kernel(feat1, feat2, Wq, bq, Wk, bk, Wv, bv, Wo, bo):
    B, C, Hs, Ws = feat1.shape
    N = Hs * Ws
    f1 = feat1.reshape(B, C, N)
    f2 = feat2.reshape(B, C, N)
    peT = _pos_encoding_t(Hs, Ws)
    bq2 = bq.reshape(-1, 1)
    bk2 = bk.reshape(-1, 1)
    bv2 = bv.reshape(-1, 1)
    bo2 = bo.reshape(-1, 1)

    grid_p = (B,)
    qkv_shape = jax.ShapeDtypeStruct((B, _TOKEN_DIM, N), jnp.float32)
    qt, kt, vt = pl.pallas_call(
        _proj_kernel,
        grid=grid_p,
        in_specs=[
            pl.BlockSpec((1, C, N), lambda b: (b, 0, 0)),
            pl.BlockSpec((1, C, N), lambda b: (b, 0, 0)),
            pl.BlockSpec((C, _TOKEN_DIM), lambda b: (0, 0)),
            pl.BlockSpec((C, _TOKEN_DIM), lambda b: (0, 0)),
            pl.BlockSpec((C, _TOKEN_DIM), lambda b: (0, 0)),
            pl.BlockSpec((_TOKEN_DIM, 1), lambda b: (0, 0)),
            pl.BlockSpec((_TOKEN_DIM, 1), lambda b: (0, 0)),
            pl.BlockSpec((_TOKEN_DIM, 1), lambda b: (0, 0)),
            pl.BlockSpec((_TOKEN_DIM, N), lambda b: (0, 0)),
        ],
        out_specs=[
            pl.BlockSpec((1, _TOKEN_DIM, N), lambda b: (b, 0, 0)),
            pl.BlockSpec((1, _TOKEN_DIM, N), lambda b: (b, 0, 0)),
            pl.BlockSpec((1, _TOKEN_DIM, N), lambda b: (b, 0, 0)),
        ],
        out_shape=[qkv_shape, qkv_shape, qkv_shape],
    )(f1, f2, Wq, Wk, Wv, bq2, bk2, bv2, peT)

    BQ = 256
    grid_a = (B, pl.cdiv(N, BQ))
    outT = pl.pallas_call(
        _attn_kernel,
        grid=grid_a,
        in_specs=[
            pl.BlockSpec((1, _TOKEN_DIM, BQ), lambda b, j: (b, 0, j)),
            pl.BlockSpec((1, _TOKEN_DIM, N), lambda b, j: (b, 0, 0)),
            pl.BlockSpec((1, _TOKEN_DIM, N), lambda b, j: (b, 0, 0)),
            pl.BlockSpec((_TOKEN_DIM, _TOKEN_DIM), lambda b, j: (0, 0)),
            pl.BlockSpec((_TOKEN_DIM, 1), lambda b, j: (0, 0)),
        ],
        out_specs=pl.BlockSpec((1, _TOKEN_DIM, BQ), lambda b, j: (b, 0, j)),
        out_shape=jax.ShapeDtypeStruct((B, _TOKEN_DIM, N), jnp.float32),
        compiler_params=pltpu.CompilerParams(
            dimension_semantics=("parallel", "parallel")),
    )(qt, kt, vt, Wo, bo2)

    return outT.reshape(B, _TOKEN_DIM, Hs, Ws)


# speedup vs baseline: 2.6686x; 1.0012x over previous
import math

import jax
import jax.numpy as jnp
from jax.experimental import pallas as pl
from jax.experimental.pallas import tpu as pltpu

_TOKEN_DIM = 192
_NUM_HEADS = 4
_HEAD_DIM = _TOKEN_DIM // _NUM_HEADS
_TOPK = 128

def _pos_encoding_t(Hs, Ws):
    y_pos = jnp.repeat(jnp.arange(Hs), Ws).astype(jnp.float32)
    x_pos = jnp.tile(jnp.arange(Ws), Hs).astype(jnp.float32)
    dim_half = _TOKEN_DIM // 2
    div_term = jnp.exp(
        jnp.arange(dim_half).astype(jnp.float32) * (-math.log(10000.0) / dim_half))
    pe_sin = jnp.sin(div_term[:, None] * x_pos[None, :])
    pe_cos = jnp.cos(div_term[: _TOKEN_DIM - dim_half, None] * y_pos[None, :])
    return jnp.concatenate([pe_sin, pe_cos], axis=0)


def _dot_t(a, b):
    return jax.lax.dot_general(
        a, b, (((0,), (0,)), ((), ())),
        preferred_element_type=jnp.float32,
        precision=jax.lax.Precision.DEFAULT)


def _dot_nt(a, b):
    return jax.lax.dot_general(
        a, b, (((1,), (1,)), ((), ())),
        preferred_element_type=jnp.float32,
        precision=jax.lax.Precision.DEFAULT)


def _proj_kernel(f1_ref, f2_ref, wq_ref, wk_ref, wv_ref,
                 bq_ref, bk_ref, bv_ref, pe_ref,
                 qt_ref, kt_ref, vt_ref):
    f1 = f1_ref[0]
    f2 = f2_ref[0]
    pe = pe_ref[...]
    qt_ref[0] = _dot_t(wq_ref[...], f1) + bq_ref[...] + pe
    kt_ref[0] = _dot_t(wk_ref[...], f2) + bk_ref[...] + pe
    vt_ref[0] = _dot_t(wv_ref[...], f2) + bv_ref[...]


def _sortable_keys(s):
    u = jax.lax.bitcast_convert_type(s, jnp.int32)
    return jnp.where(u >= 0, u, jnp.full_like(u, -2147483648) - u)


def _topk_mask(s, k):
    keys = _sortable_keys(s)
    rows = s.shape[0]
    lo = jnp.full((rows, 1), -2147483648, dtype=jnp.int32)
    hi = jnp.full((rows, 1), 2147483647, dtype=jnp.int32)

    def body(_, c):
        lo, hi = c
        mid = (lo >> 1) + (hi >> 1) + ((lo | hi) & 1)
        cnt = jnp.sum(keys >= mid, axis=-1, keepdims=True)
        ge = cnt >= k
        return jnp.where(ge, mid, lo), jnp.where(ge, hi, mid - 1)

    lo, hi = jax.lax.fori_loop(0, 32, body, (lo, hi))
    return keys >= lo


def _attn_kernel(qt_ref, kt_ref, vt_ref, wo_ref, bo_ref, out_ref):
    scale = 1.0 / math.sqrt(_HEAD_DIM)
    bq = qt_ref.shape[2]
    s_parts = []
    for h in range(_NUM_HEADS):
        sl = slice(h * _HEAD_DIM, (h + 1) * _HEAD_DIM)
        q = qt_ref[0, sl, :]
        kk = kt_ref[0, sl, :]
        s_parts.append(_dot_t(q, kk) * scale)

    s = jnp.concatenate(s_parts, axis=0)
    mask = _topk_mask(s, _TOPK)
    rowmax = jnp.max(s, axis=-1, keepdims=True)
    w = jnp.where(mask, jnp.exp(s - rowmax), 0.0)
    denom = jnp.sum(w, axis=-1, keepdims=True)
    attn = w * (1.0 / denom)

    ctx_parts = []
    for h in range(_NUM_HEADS):
        sl = slice(h * _HEAD_DIM, (h + 1) * _HEAD_DIM)
        v = vt_ref[0, sl, :]
        ctx_parts.append(_dot_nt(v, attn[h * bq:(h + 1) * bq]))

    ctxT = jnp.concatenate(ctx_parts, axis=0)
    out_ref[0] = _dot_t(wo_ref[...], ctxT) + bo_ref[...]


def kernel(feat1, feat2, Wq, bq, Wk, bk, Wv, bv, Wo, bo):
    B, C, Hs, Ws = feat1.shape
    N = Hs * Ws
    f1 = feat1.reshape(B, C, N)
    f2 = feat2.reshape(B, C, N)
    peT = _pos_encoding_t(Hs, Ws)
    bq2 = bq.reshape(-1, 1)
    bk2 = bk.reshape(-1, 1)
    bv2 = bv.reshape(-1, 1)
    bo2 = bo.reshape(-1, 1)

    grid_p = (B,)
    qkv_shape = jax.ShapeDtypeStruct((B, _TOKEN_DIM, N), jnp.float32)
    qt, kt, vt = pl.pallas_call(
        _proj_kernel,
        grid=grid_p,
        in_specs=[
            pl.BlockSpec((1, C, N), lambda b: (b, 0, 0)),
            pl.BlockSpec((1, C, N), lambda b: (b, 0, 0)),
            pl.BlockSpec((C, _TOKEN_DIM), lambda b: (0, 0)),
            pl.BlockSpec((C, _TOKEN_DIM), lambda b: (0, 0)),
            pl.BlockSpec((C, _TOKEN_DIM), lambda b: (0, 0)),
            pl.BlockSpec((_TOKEN_DIM, 1), lambda b: (0, 0)),
            pl.BlockSpec((_TOKEN_DIM, 1), lambda b: (0, 0)),
            pl.BlockSpec((_TOKEN_DIM, 1), lambda b: (0, 0)),
            pl.BlockSpec((_TOKEN_DIM, N), lambda b: (0, 0)),
        ],
        out_specs=[
            pl.BlockSpec((1, _TOKEN_DIM, N), lambda b: (b, 0, 0)),
            pl.BlockSpec((1, _TOKEN_DIM, N), lambda b: (b, 0, 0)),
            pl.BlockSpec((1, _TOKEN_DIM, N), lambda b: (b, 0, 0)),
        ],
        out_shape=[qkv_shape, qkv_shape, qkv_shape],
    )(f1, f2, Wq, Wk, Wv, bq2, bk2, bv2, peT)

    BQ = 256
    grid_a = (B, pl.cdiv(N, BQ))
    outT = pl.pallas_call(
        _attn_kernel,
        grid=grid_a,
        in_specs=[
            pl.BlockSpec((1, _TOKEN_DIM, BQ), lambda b, j: (b, 0, j)),
            pl.BlockSpec((1, _TOKEN_DIM, N), lambda b, j: (b, 0, 0)),
            pl.BlockSpec((1, _TOKEN_DIM, N), lambda b, j: (b, 0, 0)),
            pl.BlockSpec((_TOKEN_DIM, _TOKEN_DIM), lambda b, j: (0, 0)),
            pl.BlockSpec((_TOKEN_DIM, 1), lambda b, j: (0, 0)),
        ],
        out_specs=pl.BlockSpec((1, _TOKEN_DIM, BQ), lambda b, j: (b, 0, j)),
        out_shape=jax.ShapeDtypeStruct((B, _TOKEN_DIM, N), jnp.float32),
        compiler_params=pltpu.CompilerParams(
            dimension_semantics=("parallel", "parallel")),
    )(qt, kt, vt, Wo, bo2)

    return outT.reshape(B, _TOKEN_DIM, Hs, Ws)
